# baseline (speedup 1.0000x reference)
import functools

import jax
import jax.numpy as jnp
from jax import lax
from jax.experimental import pallas as pl
from jax.experimental.pallas import tpu as pltpu
from jax.experimental.pallas import tpu_sc as plsc

N = 10000
NT = 10240
DW = 144
D = 128
NC = 2
NS = 16
NW = NC * NS
E = 320000
CH = 64
NB = 4
NCH = 160
EPW = NCH * CH
EPAD = NW * EPW
RPT = NT // NS
BLK = 1024


def _sc_scatter_sum(table, src_idx, dst_idx, W):
    mesh = plsc.VectorSubcoreMesh(core_axis_name="c", subcore_axis_name="s",
                                  num_cores=NC, num_subcores=NS)

    @functools.partial(
        pl.kernel,
        out_type=jax.ShapeDtypeStruct((NC * NT, W), jnp.float32),
        mesh=mesh,
        scratch_types=[
            pltpu.VMEM((NB, CH), jnp.int32),
            pltpu.VMEM((NB, CH), jnp.int32),
            pltpu.VMEM((NB, CH, W), jnp.float32),
            pltpu.VMEM_SHARED((NT, W), jnp.float32),
            [pltpu.SemaphoreType.DMA] * NB,
            [pltpu.SemaphoreType.DMA] * NB,
            [pltpu.SemaphoreType.DMA] * NB,
        ],
        compiler_params=pltpu.CompilerParams(use_tc_tiling_on_sc=False),
    )
    def k(table_hbm, src_hbm, dst_hbm, out_hbm,
          sidx_r, didx_r, rows_v, acc_sh, sem_i, sem_g, sem_s):
        c = lax.axis_index("c")
        s = lax.axis_index("s")
        wid = s * NC + c

        def idx_load(j, b):
            pltpu.async_copy(src_hbm.at[wid, j], sidx_r.at[b], sem_i[b])
            pltpu.async_copy(dst_hbm.at[wid, j], didx_r.at[b], sem_i[b])

        def idx_wait(j, b):
            for _ in range(2):
                pltpu.make_async_copy(
                    src_hbm.at[wid, j], sidx_r.at[b], sem_i[b]).wait()

        def gather_start(b):
            pltpu.async_copy(table_hbm.at[sidx_r.at[b]], rows_v.at[b], sem_g[b])

        def gather_wait(b):
            pltpu.make_async_copy(
                table_hbm.at[sidx_r.at[b]], rows_v.at[b], sem_g[b]).wait()

        def scatter_start(b):
            pltpu.async_copy(rows_v.at[b], acc_sh.at[didx_r.at[b]], sem_s[b],
                             add=True)

        def scatter_wait(b):
            pltpu.make_async_copy(
                rows_v.at[b], acc_sh.at[didx_r.at[b]], sem_s[b]).wait()

        zero16 = jnp.zeros((16,), jnp.float32)

        def zrow(i, _):
            for j in range(W // 16):
                rows_v[0, i, pl.ds(j * 16, 16)] = zero16
            return 0

        lax.fori_loop(0, CH, zrow, 0)
        for r in range(RPT // CH):
            pltpu.sync_copy(rows_v.at[0], acc_sh.at[pl.ds(s * RPT + r * CH, CH)])
        plsc.subcore_barrier()

        idx_load(0, 0)
        idx_load(1, 1)
        idx_wait(0, 0)
        gather_start(0)

        def step(i, _):
            for b in range(NB):
                j = i * NB + b
                b1 = (b + 1) % NB
                b2 = (b + 2) % NB

                @pl.when(j >= 2)
                def _():
                    scatter_wait(b2)

                @pl.when(j + 2 < NCH)
                def _():
                    idx_load(j + 2, b2)

                @pl.when(j + 1 < NCH)
                def _():
                    idx_wait(j + 1, b1)
                    gather_start(b1)

                gather_wait(b)
                scatter_start(b)
            return 0

        lax.fori_loop(0, NCH // NB, step, 0)
        scatter_wait((NCH - 2) % NB)
        scatter_wait((NCH - 1) % NB)
        plsc.subcore_barrier()

        pltpu.sync_copy(acc_sh.at[pl.ds(s * RPT, RPT)],
                        out_hbm.at[pl.ds(c * NT + s * RPT, RPT)])

    return k(table, src_idx, dst_idx)


def _tc_layer1(x_aug, p0, p1, Wlt, Wrt, b):
    def body(h_ref, p0_ref, p1_ref, wl_ref, wr_ref, b_ref, o_ref, oc_ref):
        i = pl.program_id(0)
        hb = h_ref[...]
        ssum = p0_ref[...] + p1_ref[...]
        inv_c = 1.0 / jnp.maximum(ssum[:, D:D + 1], 1.0)
        mean = ssum[:, :D] * inv_c
        xr = hb[:, :D]
        z = (jnp.dot(mean, wl_ref[...], preferred_element_type=jnp.float32)
             + jnp.dot(xr, wr_ref[...], preferred_element_type=jnp.float32)
             + b_ref[...])
        nrm = jnp.sqrt(jnp.sum(z * z, axis=1, keepdims=True))
        hn = jnp.maximum(z / jnp.maximum(nrm, 1e-12), 0.0)
        row = i * BLK + lax.broadcasted_iota(jnp.int32, (BLK, 1), 0)
        valid = row < N
        o_ref[...] = jnp.where(valid, hn, 0.0)
        oc_ref[...] = jnp.where(valid, inv_c, 0.0)

    return pl.pallas_call(
        body,
        grid=(NT // BLK,),
        in_specs=[
            pl.BlockSpec((BLK, DW), lambda i: (i, 0)),
            pl.BlockSpec((BLK, DW), lambda i: (i, 0)),
            pl.BlockSpec((BLK, DW), lambda i: (i, 0)),
            pl.BlockSpec((D, D), lambda i: (0, 0)),
            pl.BlockSpec((D, D), lambda i: (0, 0)),
            pl.BlockSpec((1, D), lambda i: (0, 0)),
        ],
        out_specs=[pl.BlockSpec((BLK, D), lambda i: (i, 0)),
                   pl.BlockSpec((BLK, 1), lambda i: (i, 0))],
        out_shape=[jax.ShapeDtypeStruct((NT, D), jnp.float32),
                   jax.ShapeDtypeStruct((NT, 1), jnp.float32)],
    )(x_aug, p0, p1, Wlt, Wrt, b)


def _tc_layer2(h, p0, p1, invc, Wlt, Wrt, b):
    def body(h_ref, p0_ref, p1_ref, c_ref, wl_ref, wr_ref, b_ref, o_ref):
        i = pl.program_id(0)
        mean = (p0_ref[...] + p1_ref[...]) * c_ref[...]
        z = (jnp.dot(mean, wl_ref[...], preferred_element_type=jnp.float32)
             + jnp.dot(h_ref[...], wr_ref[...], preferred_element_type=jnp.float32)
             + b_ref[...])
        nrm = jnp.sqrt(jnp.sum(z * z, axis=1, keepdims=True))
        hn = jnp.maximum(z / jnp.maximum(nrm, 1e-12), 0.0)
        row = i * BLK + lax.broadcasted_iota(jnp.int32, (BLK, 1), 0)
        o_ref[...] = jnp.where(row < N, hn, 0.0)

    return pl.pallas_call(
        body,
        grid=(NT // BLK,),
        in_specs=[
            pl.BlockSpec((BLK, D), lambda i: (i, 0)),
            pl.BlockSpec((BLK, D), lambda i: (i, 0)),
            pl.BlockSpec((BLK, D), lambda i: (i, 0)),
            pl.BlockSpec((BLK, 1), lambda i: (i, 0)),
            pl.BlockSpec((D, D), lambda i: (0, 0)),
            pl.BlockSpec((D, D), lambda i: (0, 0)),
            pl.BlockSpec((1, D), lambda i: (0, 0)),
        ],
        out_specs=pl.BlockSpec((BLK, D), lambda i: (i, 0)),
        out_shape=jax.ShapeDtypeStruct((NT, D), jnp.float32),
    )(h, p0, p1, invc, Wlt, Wrt, b)


def _tc_head(h, p0, p1, invc, W3lt, W3rt, b3, Wit, bi2, Wct, bc2, IH):
    def body(h_ref, p0_ref, p1_ref, c_ref, wl_ref, wr_ref, b3_ref, wi_ref,
             bi_ref, wc_ref, bc_ref, o_ref):
        mean = (p0_ref[...] + p1_ref[...]) * c_ref[...]
        z = (jnp.dot(mean, wl_ref[...], preferred_element_type=jnp.float32)
             + jnp.dot(h_ref[...], wr_ref[...], preferred_element_type=jnp.float32)
             + b3_ref[...])
        nrm = jnp.sqrt(jnp.sum(z * z, axis=1, keepdims=True))
        h3 = jnp.maximum(z / jnp.maximum(nrm, 1e-12), 0.0)
        h4 = jnp.maximum(
            jnp.dot(h3, wi_ref[...], preferred_element_type=jnp.float32)
            + bi_ref[...], 0.0)
        o_ref[...] = (jnp.dot(h4, wc_ref[...], preferred_element_type=jnp.float32)
                      + bc_ref[...])

    return pl.pallas_call(
        body,
        grid=(NT // BLK,),
        in_specs=[
            pl.BlockSpec((BLK, D), lambda i: (i, 0)),
            pl.BlockSpec((BLK, D), lambda i: (i, 0)),
            pl.BlockSpec((BLK, D), lambda i: (i, 0)),
            pl.BlockSpec((BLK, 1), lambda i: (i, 0)),
            pl.BlockSpec((D, IH), lambda i: (0, 0)),
            pl.BlockSpec((D, IH), lambda i: (0, 0)),
            pl.BlockSpec((1, IH), lambda i: (0, 0)),
            pl.BlockSpec((IH, IH), lambda i: (0, 0)),
            pl.BlockSpec((1, IH), lambda i: (0, 0)),
            pl.BlockSpec((IH, D), lambda i: (0, 0)),
            pl.BlockSpec((1, D), lambda i: (0, 0)),
        ],
        out_specs=pl.BlockSpec((BLK, D), lambda i: (i, 0)),
        out_shape=jax.ShapeDtypeStruct((NT, D), jnp.float32),
    )(h, p0, p1, invc, W3lt, W3rt, b3, Wit, bi2, Wct, bc2)


def kernel(x, edge_index, batch, W1l, b1l, W1r, b1r, W2l, b2l, W2r, b2r,
           W3l, b3l, W3r, b3r, Wi, bi, Wc, bc):
    IH = Wi.shape[0]
    O = Wc.shape[0]

    x_aug = jnp.zeros((NT, DW), jnp.float32)
    x_aug = x_aug.at[:N, :D].set(x).at[:N, D].set(1.0)

    pad = N + jnp.arange(EPAD - E, dtype=jnp.int32) % (NT - N)
    srcp = jnp.concatenate([edge_index[0], pad]).reshape(NW, NCH, CH)
    dstp = jnp.concatenate([edge_index[1], pad]).reshape(NW, NCH, CH)

    def partials(tab, W):
        P = _sc_scatter_sum(tab, srcp, dstp, W)
        return P[:NT], P[NT:]

    p0, p1 = partials(x_aug, DW)
    h1, invc = _tc_layer1(x_aug, p0, p1, W1l.T, W1r.T, (b1l + b1r)[None, :])
    p0, p1 = partials(h1, D)
    h2 = _tc_layer2(h1, p0, p1, invc, W2l.T, W2r.T, (b2l + b2r)[None, :])
    p0, p1 = partials(h2, D)

    Wct = jnp.zeros((D, IH), jnp.float32).at[:O].set(Wc).T
    bc2 = jnp.zeros((1, D), jnp.float32).at[0, :O].set(bc)
    out = _tc_head(h2, p0, p1, invc, W3l.T, W3r.T, (b3l + b3r)[None, :],
                   Wi.T, bi[None, :], Wct, bc2, IH)
    return out[:N, :O]


# speedup vs baseline: 1.0369x; 1.0369x over previous
import functools

import jax
import jax.numpy as jnp
from jax import lax
from jax.experimental import pallas as pl
from jax.experimental.pallas import tpu as pltpu
from jax.experimental.pallas import tpu_sc as plsc

N = 10000
NT = 10240
DW = 144
D = 128
NC = 2
NS = 16
NW = NC * NS
E = 320000
CH = 64
NB = 4
NCH = 160
EPW = NCH * CH
EPAD = NW * EPW
RPT = NT // NS
BLK = 1024


def _sc_scatter_sum(table, src_idx, dst_idx, W):
    mesh = plsc.VectorSubcoreMesh(core_axis_name="c", subcore_axis_name="s",
                                  num_cores=NC, num_subcores=NS)

    @functools.partial(
        pl.kernel,
        out_type=jax.ShapeDtypeStruct((NC * NT, W), jnp.float32),
        mesh=mesh,
        scratch_types=[
            pltpu.VMEM((NB, CH), jnp.int32),
            pltpu.VMEM((NB, CH), jnp.int32),
            pltpu.VMEM((NB, CH, W), jnp.float32),
            pltpu.VMEM_SHARED((NT, W), jnp.float32),
            [pltpu.SemaphoreType.DMA] * NB,
            [pltpu.SemaphoreType.DMA] * NB,
            [pltpu.SemaphoreType.DMA] * NB,
        ],
        compiler_params=pltpu.CompilerParams(use_tc_tiling_on_sc=False),
    )
    def k(table_hbm, src_hbm, dst_hbm, out_hbm,
          sidx_r, didx_r, rows_v, acc_sh, sem_i, sem_g, sem_s):
        c = lax.axis_index("c")
        s = lax.axis_index("s")
        wid = s * NC + c

        def idx_load(j, b):
            pltpu.async_copy(src_hbm.at[wid, j], sidx_r.at[b], sem_i[b])
            pltpu.async_copy(dst_hbm.at[wid, j], didx_r.at[b], sem_i[b])

        def idx_wait(j, b):
            for _ in range(2):
                pltpu.make_async_copy(
                    src_hbm.at[wid, j], sidx_r.at[b], sem_i[b]).wait()

        def gather_start(b):
            pltpu.async_copy(table_hbm.at[sidx_r.at[b]], rows_v.at[b], sem_g[b])

        def gather_wait(b):
            pltpu.make_async_copy(
                table_hbm.at[sidx_r.at[b]], rows_v.at[b], sem_g[b]).wait()

        def scatter_start(b):
            pltpu.async_copy(rows_v.at[b], acc_sh.at[didx_r.at[b]], sem_s[b],
                             add=True)

        def scatter_wait(b):
            pltpu.make_async_copy(
                rows_v.at[b], acc_sh.at[didx_r.at[b]], sem_s[b]).wait()

        idx_load(0, 0)
        idx_load(1, 1)
        zero16 = jnp.zeros((16,), jnp.float32)
        zb = NB - 1

        def zrow(i, _):
            for j in range(W // 16):
                rows_v[zb, i, pl.ds(j * 16, 16)] = zero16
            return 0

        lax.fori_loop(0, CH, zrow, 0)
        idx_wait(0, 0)
        gather_start(0)
        for r in range(RPT // CH):
            pltpu.async_copy(rows_v.at[zb],
                             acc_sh.at[pl.ds(s * RPT + r * CH, CH)], sem_s[zb])
        for r in range(RPT // CH):
            pltpu.make_async_copy(
                rows_v.at[zb],
                acc_sh.at[pl.ds(s * RPT + r * CH, CH)], sem_s[zb]).wait()
        plsc.subcore_barrier()

        def step(i, _):
            for b in range(NB):
                j = i * NB + b
                b1 = (b + 1) % NB
                b2 = (b + 2) % NB

                @pl.when(j >= 2)
                def _():
                    scatter_wait(b2)

                @pl.when(j + 2 < NCH)
                def _():
                    idx_load(j + 2, b2)

                @pl.when(j + 1 < NCH)
                def _():
                    idx_wait(j + 1, b1)
                    gather_start(b1)

                gather_wait(b)
                scatter_start(b)
            return 0

        lax.fori_loop(0, NCH // NB, step, 0)
        scatter_wait((NCH - 2) % NB)
        scatter_wait((NCH - 1) % NB)
        plsc.subcore_barrier()

        pltpu.sync_copy(acc_sh.at[pl.ds(s * RPT, RPT)],
                        out_hbm.at[pl.ds(c * NT + s * RPT, RPT)])

    return k(table, src_idx, dst_idx)


def _tc_layer1(x_aug, p0, p1, Wlt, Wrt, b):
    def body(h_ref, p0_ref, p1_ref, wl_ref, wr_ref, b_ref, o_ref, oc_ref):
        i = pl.program_id(0)
        hb = h_ref[...]
        ssum = p0_ref[...] + p1_ref[...]
        inv_c = 1.0 / jnp.maximum(ssum[:, D:D + 1], 1.0)
        mean = ssum[:, :D] * inv_c
        xr = hb[:, :D]
        z = (jnp.dot(mean, wl_ref[...], preferred_element_type=jnp.float32)
             + jnp.dot(xr, wr_ref[...], preferred_element_type=jnp.float32)
             + b_ref[...])
        nrm = jnp.sqrt(jnp.sum(z * z, axis=1, keepdims=True))
        hn = jnp.maximum(z / jnp.maximum(nrm, 1e-12), 0.0)
        row = i * BLK + lax.broadcasted_iota(jnp.int32, (BLK, 1), 0)
        valid = row < N
        o_ref[...] = jnp.where(valid, hn, 0.0)
        oc_ref[...] = jnp.where(valid, inv_c, 0.0)

    return pl.pallas_call(
        body,
        grid=(NT // BLK,),
        in_specs=[
            pl.BlockSpec((BLK, DW), lambda i: (i, 0)),
            pl.BlockSpec((BLK, DW), lambda i: (i, 0)),
            pl.BlockSpec((BLK, DW), lambda i: (i, 0)),
            pl.BlockSpec((D, D), lambda i: (0, 0)),
            pl.BlockSpec((D, D), lambda i: (0, 0)),
            pl.BlockSpec((1, D), lambda i: (0, 0)),
        ],
        out_specs=[pl.BlockSpec((BLK, D), lambda i: (i, 0)),
                   pl.BlockSpec((BLK, 1), lambda i: (i, 0))],
        out_shape=[jax.ShapeDtypeStruct((NT, D), jnp.float32),
                   jax.ShapeDtypeStruct((NT, 1), jnp.float32)],
    )(x_aug, p0, p1, Wlt, Wrt, b)


def _tc_layer2(h, p0, p1, invc, Wlt, Wrt, b):
    def body(h_ref, p0_ref, p1_ref, c_ref, wl_ref, wr_ref, b_ref, o_ref):
        i = pl.program_id(0)
        mean = (p0_ref[...] + p1_ref[...]) * c_ref[...]
        z = (jnp.dot(mean, wl_ref[...], preferred_element_type=jnp.float32)
             + jnp.dot(h_ref[...], wr_ref[...], preferred_element_type=jnp.float32)
             + b_ref[...])
        nrm = jnp.sqrt(jnp.sum(z * z, axis=1, keepdims=True))
        hn = jnp.maximum(z / jnp.maximum(nrm, 1e-12), 0.0)
        row = i * BLK + lax.broadcasted_iota(jnp.int32, (BLK, 1), 0)
        o_ref[...] = jnp.where(row < N, hn, 0.0)

    return pl.pallas_call(
        body,
        grid=(NT // BLK,),
        in_specs=[
            pl.BlockSpec((BLK, D), lambda i: (i, 0)),
            pl.BlockSpec((BLK, D), lambda i: (i, 0)),
            pl.BlockSpec((BLK, D), lambda i: (i, 0)),
            pl.BlockSpec((BLK, 1), lambda i: (i, 0)),
            pl.BlockSpec((D, D), lambda i: (0, 0)),
            pl.BlockSpec((D, D), lambda i: (0, 0)),
            pl.BlockSpec((1, D), lambda i: (0, 0)),
        ],
        out_specs=pl.BlockSpec((BLK, D), lambda i: (i, 0)),
        out_shape=jax.ShapeDtypeStruct((NT, D), jnp.float32),
    )(h, p0, p1, invc, Wlt, Wrt, b)


def _tc_head(h, p0, p1, invc, W3lt, W3rt, b3, Wit, bi2, Wct, bc2, IH):
    def body(h_ref, p0_ref, p1_ref, c_ref, wl_ref, wr_ref, b3_ref, wi_ref,
             bi_ref, wc_ref, bc_ref, o_ref):
        mean = (p0_ref[...] + p1_ref[...]) * c_ref[...]
        z = (jnp.dot(mean, wl_ref[...], preferred_element_type=jnp.float32)
             + jnp.dot(h_ref[...], wr_ref[...], preferred_element_type=jnp.float32)
             + b3_ref[...])
        nrm = jnp.sqrt(jnp.sum(z * z, axis=1, keepdims=True))
        h3 = jnp.maximum(z / jnp.maximum(nrm, 1e-12), 0.0)
        h4 = jnp.maximum(
            jnp.dot(h3, wi_ref[...], preferred_element_type=jnp.float32)
            + bi_ref[...], 0.0)
        o_ref[...] = (jnp.dot(h4, wc_ref[...], preferred_element_type=jnp.float32)
                      + bc_ref[...])

    return pl.pallas_call(
        body,
        grid=(NT // BLK,),
        in_specs=[
            pl.BlockSpec((BLK, D), lambda i: (i, 0)),
            pl.BlockSpec((BLK, D), lambda i: (i, 0)),
            pl.BlockSpec((BLK, D), lambda i: (i, 0)),
            pl.BlockSpec((BLK, 1), lambda i: (i, 0)),
            pl.BlockSpec((D, IH), lambda i: (0, 0)),
            pl.BlockSpec((D, IH), lambda i: (0, 0)),
            pl.BlockSpec((1, IH), lambda i: (0, 0)),
            pl.BlockSpec((IH, IH), lambda i: (0, 0)),
            pl.BlockSpec((1, IH), lambda i: (0, 0)),
            pl.BlockSpec((IH, D), lambda i: (0, 0)),
            pl.BlockSpec((1, D), lambda i: (0, 0)),
        ],
        out_specs=pl.BlockSpec((BLK, D), lambda i: (i, 0)),
        out_shape=jax.ShapeDtypeStruct((NT, D), jnp.float32),
    )(h, p0, p1, invc, W3lt, W3rt, b3, Wit, bi2, Wct, bc2)


def kernel(x, edge_index, batch, W1l, b1l, W1r, b1r, W2l, b2l, W2r, b2r,
           W3l, b3l, W3r, b3r, Wi, bi, Wc, bc):
    IH = Wi.shape[0]
    O = Wc.shape[0]

    x_aug = jnp.zeros((NT, DW), jnp.float32)
    x_aug = x_aug.at[:N, :D].set(x).at[:N, D].set(1.0)

    pad = N + jnp.arange(EPAD - E, dtype=jnp.int32) % (NT - N)
    srcp = jnp.concatenate([edge_index[0], pad]).reshape(NW, NCH, CH)
    dstp = jnp.concatenate([edge_index[1], pad]).reshape(NW, NCH, CH)

    def partials(tab, W):
        P = _sc_scatter_sum(tab, srcp, dstp, W)
        return P[:NT], P[NT:]

    p0, p1 = partials(x_aug, DW)
    h1, invc = _tc_layer1(x_aug, p0, p1, W1l.T, W1r.T, (b1l + b1r)[None, :])
    p0, p1 = partials(h1, D)
    h2 = _tc_layer2(h1, p0, p1, invc, W2l.T, W2r.T, (b2l + b2r)[None, :])
    p0, p1 = partials(h2, D)

    Wct = jnp.zeros((D, IH), jnp.float32).at[:O].set(Wc).T
    bc2 = jnp.zeros((1, D), jnp.float32).at[0, :O].set(bc)
    out = _tc_head(h2, p0, p1, invc, W3l.T, W3r.T, (b3l + b3r)[None, :],
                   Wi.T, bi[None, :], Wct, bc2, IH)
    return out[:N, :O]


# speedup vs baseline: 1.0986x; 1.0595x over previous
import functools

import jax
import jax.numpy as jnp
from jax import lax
from jax.experimental import pallas as pl
from jax.experimental.pallas import tpu as pltpu
from jax.experimental.pallas import tpu_sc as plsc

N = 10000
NT = 10240
DW = 144
D = 128
NC = 2
NS = 16
NW = NC * NS
E = 320000
CH = 64
NB = 4
NCH = 160
EPW = NCH * CH
EPAD = NW * EPW
RPT = NT // NS
BLK = 1024


def _sc_scatter_sum(table, src_idx, dst_idx, W):
    mesh = plsc.VectorSubcoreMesh(core_axis_name="c", subcore_axis_name="s",
                                  num_cores=NC, num_subcores=NS)

    @functools.partial(
        pl.kernel,
        out_type=jax.ShapeDtypeStruct((NC * NT, W), jnp.float32),
        mesh=mesh,
        scratch_types=[
            pltpu.VMEM((NB, CH), jnp.int32),
            pltpu.VMEM((NB, CH), jnp.int32),
            pltpu.VMEM((NB, CH, W), jnp.float32),
            pltpu.VMEM_SHARED((NT, W), jnp.float32),
            [pltpu.SemaphoreType.DMA] * NB,
            [pltpu.SemaphoreType.DMA] * NB,
            [pltpu.SemaphoreType.DMA] * NB,
        ],
        compiler_params=pltpu.CompilerParams(use_tc_tiling_on_sc=False),
    )
    def k(table_hbm, src_hbm, dst_hbm, out_hbm,
          sidx_r, didx_r, rows_v, acc_sh, sem_i, sem_g, sem_s):
        c = lax.axis_index("c")
        s = lax.axis_index("s")
        wid = s * NC + c

        def idx_load(j, b):
            pltpu.async_copy(src_hbm.at[wid, j], sidx_r.at[b], sem_i[b])
            pltpu.async_copy(dst_hbm.at[wid, j], didx_r.at[b], sem_i[b])

        def idx_wait(j, b):
            for _ in range(2):
                pltpu.make_async_copy(
                    src_hbm.at[wid, j], sidx_r.at[b], sem_i[b]).wait()

        def gather_start(b):
            pltpu.async_copy(table_hbm.at[sidx_r.at[b]], rows_v.at[b], sem_g[b])

        def gather_wait(b):
            pltpu.make_async_copy(
                table_hbm.at[sidx_r.at[b]], rows_v.at[b], sem_g[b]).wait()

        def scatter_start(b):
            pltpu.async_copy(rows_v.at[b], acc_sh.at[didx_r.at[b]], sem_s[b],
                             add=True)

        def scatter_wait(b):
            pltpu.make_async_copy(
                rows_v.at[b], acc_sh.at[didx_r.at[b]], sem_s[b]).wait()

        idx_load(0, 0)
        idx_load(1, 1)
        zero16 = jnp.zeros((16,), jnp.float32)
        zb = NB - 1

        def zrow(i, _):
            for j in range(W // 16):
                rows_v[zb, i, pl.ds(j * 16, 16)] = zero16
            return 0

        lax.fori_loop(0, CH, zrow, 0)
        idx_wait(0, 0)
        gather_start(0)
        for r in range(RPT // CH):
            pltpu.async_copy(rows_v.at[zb],
                             acc_sh.at[pl.ds(s * RPT + r * CH, CH)], sem_s[zb])
        for r in range(RPT // CH):
            pltpu.make_async_copy(
                rows_v.at[zb],
                acc_sh.at[pl.ds(s * RPT + r * CH, CH)], sem_s[zb]).wait()
        plsc.subcore_barrier()

        def step(i, _):
            for b in range(NB):
                j = i * NB + b
                b1 = (b + 1) % NB
                b2 = (b + 2) % NB

                @pl.when(j >= 2)
                def _():
                    scatter_wait(b2)

                @pl.when(j + 2 < NCH)
                def _():
                    idx_load(j + 2, b2)

                @pl.when(j + 1 < NCH)
                def _():
                    idx_wait(j + 1, b1)
                    gather_start(b1)

                gather_wait(b)
                scatter_start(b)
            return 0

        lax.fori_loop(0, NCH // NB, step, 0)
        scatter_wait((NCH - 2) % NB)
        scatter_wait((NCH - 1) % NB)
        plsc.subcore_barrier()

        pltpu.sync_copy(acc_sh.at[pl.ds(s * RPT, RPT)],
                        out_hbm.at[pl.ds(c * NT + s * RPT, RPT)])

    return k(table, src_idx, dst_idx)


def _tc_layer1(x_aug, P, Wlt, Wrt, b):
    def body(h_ref, p0_ref, p1_ref, wl_ref, wr_ref, b_ref, o_ref, oc_ref):
        i = pl.program_id(0)
        hb = h_ref[...]
        ssum = p0_ref[...] + p1_ref[...]
        inv_c = 1.0 / jnp.maximum(ssum[:, D:D + 1], 1.0)
        mean = ssum[:, :D] * inv_c
        xr = hb[:, :D]
        z = (jnp.dot(mean, wl_ref[...], preferred_element_type=jnp.float32)
             + jnp.dot(xr, wr_ref[...], preferred_element_type=jnp.float32)
             + b_ref[...])
        nrm = jnp.sqrt(jnp.sum(z * z, axis=1, keepdims=True))
        hn = jnp.maximum(z / jnp.maximum(nrm, 1e-12), 0.0)
        row = i * BLK + lax.broadcasted_iota(jnp.int32, (BLK, 1), 0)
        valid = row < N
        o_ref[...] = jnp.where(valid, hn, 0.0)
        oc_ref[...] = jnp.where(valid, inv_c, 0.0)

    return pl.pallas_call(
        body,
        grid=(NT // BLK,),
        in_specs=[
            pl.BlockSpec((BLK, DW), lambda i: (i, 0)),
            pl.BlockSpec((BLK, DW), lambda i: (i, 0)),
            pl.BlockSpec((BLK, DW), lambda i: (i + NT // BLK, 0)),
            pl.BlockSpec((D, D), lambda i: (0, 0)),
            pl.BlockSpec((D, D), lambda i: (0, 0)),
            pl.BlockSpec((1, D), lambda i: (0, 0)),
        ],
        out_specs=[pl.BlockSpec((BLK, D), lambda i: (i, 0)),
                   pl.BlockSpec((BLK, 1), lambda i: (i, 0))],
        out_shape=[jax.ShapeDtypeStruct((NT, D), jnp.float32),
                   jax.ShapeDtypeStruct((NT, 1), jnp.float32)],
    )(x_aug, P, P, Wlt, Wrt, b)


def _tc_layer2(h, P, invc, Wlt, Wrt, b):
    def body(h_ref, p0_ref, p1_ref, c_ref, wl_ref, wr_ref, b_ref, o_ref):
        i = pl.program_id(0)
        mean = (p0_ref[...] + p1_ref[...]) * c_ref[...]
        z = (jnp.dot(mean, wl_ref[...], preferred_element_type=jnp.float32)
             + jnp.dot(h_ref[...], wr_ref[...], preferred_element_type=jnp.float32)
             + b_ref[...])
        nrm = jnp.sqrt(jnp.sum(z * z, axis=1, keepdims=True))
        hn = jnp.maximum(z / jnp.maximum(nrm, 1e-12), 0.0)
        row = i * BLK + lax.broadcasted_iota(jnp.int32, (BLK, 1), 0)
        o_ref[...] = jnp.where(row < N, hn, 0.0)

    return pl.pallas_call(
        body,
        grid=(NT // BLK,),
        in_specs=[
            pl.BlockSpec((BLK, D), lambda i: (i, 0)),
            pl.BlockSpec((BLK, D), lambda i: (i, 0)),
            pl.BlockSpec((BLK, D), lambda i: (i + NT // BLK, 0)),
            pl.BlockSpec((BLK, 1), lambda i: (i, 0)),
            pl.BlockSpec((D, D), lambda i: (0, 0)),
            pl.BlockSpec((D, D), lambda i: (0, 0)),
            pl.BlockSpec((1, D), lambda i: (0, 0)),
        ],
        out_specs=pl.BlockSpec((BLK, D), lambda i: (i, 0)),
        out_shape=jax.ShapeDtypeStruct((NT, D), jnp.float32),
    )(h, P, P, invc, Wlt, Wrt, b)


def _tc_head(h, P, invc, W3lt, W3rt, b3, Wit, bi2, Wct, bc2, IH):
    def body(h_ref, p0_ref, p1_ref, c_ref, wl_ref, wr_ref, b3_ref, wi_ref,
             bi_ref, wc_ref, bc_ref, o_ref):
        mean = (p0_ref[...] + p1_ref[...]) * c_ref[...]
        z = (jnp.dot(mean, wl_ref[...], preferred_element_type=jnp.float32)
             + jnp.dot(h_ref[...], wr_ref[...], preferred_element_type=jnp.float32)
             + b3_ref[...])
        nrm = jnp.sqrt(jnp.sum(z * z, axis=1, keepdims=True))
        h3 = jnp.maximum(z / jnp.maximum(nrm, 1e-12), 0.0)
        h4 = jnp.maximum(
            jnp.dot(h3, wi_ref[...], preferred_element_type=jnp.float32)
            + bi_ref[...], 0.0)
        o_ref[...] = (jnp.dot(h4, wc_ref[...], preferred_element_type=jnp.float32)
                      + bc_ref[...])

    return pl.pallas_call(
        body,
        grid=(NT // BLK,),
        in_specs=[
            pl.BlockSpec((BLK, D), lambda i: (i, 0)),
            pl.BlockSpec((BLK, D), lambda i: (i, 0)),
            pl.BlockSpec((BLK, D), lambda i: (i + NT // BLK, 0)),
            pl.BlockSpec((BLK, 1), lambda i: (i, 0)),
            pl.BlockSpec((D, IH), lambda i: (0, 0)),
            pl.BlockSpec((D, IH), lambda i: (0, 0)),
            pl.BlockSpec((1, IH), lambda i: (0, 0)),
            pl.BlockSpec((IH, IH), lambda i: (0, 0)),
            pl.BlockSpec((1, IH), lambda i: (0, 0)),
            pl.BlockSpec((IH, D), lambda i: (0, 0)),
            pl.BlockSpec((1, D), lambda i: (0, 0)),
        ],
        out_specs=pl.BlockSpec((BLK, D), lambda i: (i, 0)),
        out_shape=jax.ShapeDtypeStruct((NT, D), jnp.float32),
    )(h, P, P, invc, W3lt, W3rt, b3, Wit, bi2, Wct, bc2)


def kernel(x, edge_index, batch, W1l, b1l, W1r, b1r, W2l, b2l, W2r, b2r,
           W3l, b3l, W3r, b3r, Wi, bi, Wc, bc):
    IH = Wi.shape[0]
    O = Wc.shape[0]

    x_aug = jnp.zeros((NT, DW), jnp.float32)
    x_aug = x_aug.at[:N, :D].set(x).at[:N, D].set(1.0)

    pad = N + jnp.arange(EPAD - E, dtype=jnp.int32) % (NT - N)
    srcp = jnp.concatenate([edge_index[0], pad]).reshape(NW, NCH, CH)
    dstp = jnp.concatenate([edge_index[1], pad]).reshape(NW, NCH, CH)

    P = _sc_scatter_sum(x_aug, srcp, dstp, DW)
    h1, invc = _tc_layer1(x_aug, P, W1l.T, W1r.T, (b1l + b1r)[None, :])
    P = _sc_scatter_sum(h1, srcp, dstp, D)
    h2 = _tc_layer2(h1, P, invc, W2l.T, W2r.T, (b2l + b2r)[None, :])
    P = _sc_scatter_sum(h2, srcp, dstp, D)

    Wct = jnp.zeros((D, IH), jnp.float32).at[:O].set(Wc).T
    bc2 = jnp.zeros((1, D), jnp.float32).at[0, :O].set(bc)
    out = _tc_head(h2, P, invc, W3l.T, W3r.T, (b3l + b3r)[None, :],
                   Wi.T, bi[None, :], Wct, bc2, IH)
    return out[:N, :O]


# speedup vs baseline: 1.1364x; 1.0344x over previous
import functools

import jax
import jax.numpy as jnp
from jax import lax
from jax.experimental import pallas as pl
from jax.experimental.pallas import tpu as pltpu
from jax.experimental.pallas import tpu_sc as plsc

N = 10000
NT = 10240
DW = 144
D = 128
NC = 2
NS = 16
NW = NC * NS
E = 320000
CH = 64
NB = 4
NCH = 160
EPW = NCH * CH
EPAD = NW * EPW
RPT = NT // NS
BLK = 1024


def _sc_scatter_sum(table, src_idx, dst_idx, W, CH, NCH):
    mesh = plsc.VectorSubcoreMesh(core_axis_name="c", subcore_axis_name="s",
                                  num_cores=NC, num_subcores=NS)

    @functools.partial(
        pl.kernel,
        out_type=jax.ShapeDtypeStruct((NC * NT, W), jnp.float32),
        mesh=mesh,
        scratch_types=[
            pltpu.VMEM((NB, CH), jnp.int32),
            pltpu.VMEM((NB, CH), jnp.int32),
            pltpu.VMEM((NB, CH, W), jnp.float32),
            pltpu.VMEM_SHARED((NT, W), jnp.float32),
            [pltpu.SemaphoreType.DMA] * NB,
            [pltpu.SemaphoreType.DMA] * NB,
            [pltpu.SemaphoreType.DMA] * NB,
        ],
        compiler_params=pltpu.CompilerParams(use_tc_tiling_on_sc=False),
    )
    def k(table_hbm, src_hbm, dst_hbm, out_hbm,
          sidx_r, didx_r, rows_v, acc_sh, sem_i, sem_g, sem_s):
        c = lax.axis_index("c")
        s = lax.axis_index("s")
        wid = s * NC + c

        def idx_load(j, b):
            pltpu.async_copy(src_hbm.at[wid, j], sidx_r.at[b], sem_i[b])
            pltpu.async_copy(dst_hbm.at[wid, j], didx_r.at[b], sem_i[b])

        def idx_wait(j, b):
            for _ in range(2):
                pltpu.make_async_copy(
                    src_hbm.at[wid, j], sidx_r.at[b], sem_i[b]).wait()

        def gather_start(b):
            pltpu.async_copy(table_hbm.at[sidx_r.at[b]], rows_v.at[b], sem_g[b])

        def gather_wait(b):
            pltpu.make_async_copy(
                table_hbm.at[sidx_r.at[b]], rows_v.at[b], sem_g[b]).wait()

        def scatter_start(b):
            pltpu.async_copy(rows_v.at[b], acc_sh.at[didx_r.at[b]], sem_s[b],
                             add=True)

        def scatter_wait(b):
            pltpu.make_async_copy(
                rows_v.at[b], acc_sh.at[didx_r.at[b]], sem_s[b]).wait()

        idx_load(0, 0)
        idx_load(1, 1)
        zero16 = jnp.zeros((16,), jnp.float32)
        zb = NB - 1

        def zrow(i, _):
            for j in range(W // 16):
                rows_v[zb, i, pl.ds(j * 16, 16)] = zero16
            return 0

        lax.fori_loop(0, CH, zrow, 0)
        idx_wait(0, 0)
        gather_start(0)
        for r in range(RPT // CH):
            pltpu.async_copy(rows_v.at[zb],
                             acc_sh.at[pl.ds(s * RPT + r * CH, CH)], sem_s[zb])
        for r in range(RPT // CH):
            pltpu.make_async_copy(
                rows_v.at[zb],
                acc_sh.at[pl.ds(s * RPT + r * CH, CH)], sem_s[zb]).wait()
        plsc.subcore_barrier()

        def step(i, _):
            for b in range(NB):
                j = i * NB + b
                b1 = (b + 1) % NB
                b2 = (b + 2) % NB

                @pl.when(j >= 2)
                def _():
                    scatter_wait(b2)

                @pl.when(j + 2 < NCH)
                def _():
                    idx_load(j + 2, b2)

                @pl.when(j + 1 < NCH)
                def _():
                    idx_wait(j + 1, b1)
                    gather_start(b1)

                gather_wait(b)
                scatter_start(b)
            return 0

        lax.fori_loop(0, NCH // NB, step, 0)
        scatter_wait((NCH - 2) % NB)
        scatter_wait((NCH - 1) % NB)
        plsc.subcore_barrier()

        pltpu.sync_copy(acc_sh.at[pl.ds(s * RPT, RPT)],
                        out_hbm.at[pl.ds(c * NT + s * RPT, RPT)])

    return k(table, src_idx, dst_idx)


def _tc_layer1(x_aug, P, Wlt, Wrt, b):
    def body(h_ref, p0_ref, p1_ref, wl_ref, wr_ref, b_ref, o_ref, oc_ref):
        i = pl.program_id(0)
        hb = h_ref[...]
        ssum = p0_ref[...] + p1_ref[...]
        inv_c = 1.0 / jnp.maximum(ssum[:, D:D + 1], 1.0)
        mean = ssum[:, :D] * inv_c
        xr = hb[:, :D]
        z = (jnp.dot(mean, wl_ref[...], preferred_element_type=jnp.float32)
             + jnp.dot(xr, wr_ref[...], preferred_element_type=jnp.float32)
             + b_ref[...])
        nrm = jnp.sqrt(jnp.sum(z * z, axis=1, keepdims=True))
        hn = jnp.maximum(z / jnp.maximum(nrm, 1e-12), 0.0)
        row = i * BLK + lax.broadcasted_iota(jnp.int32, (BLK, 1), 0)
        valid = row < N
        o_ref[...] = jnp.where(valid, hn, 0.0)
        oc_ref[...] = jnp.where(valid, inv_c, 0.0)

    return pl.pallas_call(
        body,
        grid=(NT // BLK,),
        in_specs=[
            pl.BlockSpec((BLK, DW), lambda i: (i, 0)),
            pl.BlockSpec((BLK, DW), lambda i: (i, 0)),
            pl.BlockSpec((BLK, DW), lambda i: (i + NT // BLK, 0)),
            pl.BlockSpec((D, D), lambda i: (0, 0)),
            pl.BlockSpec((D, D), lambda i: (0, 0)),
            pl.BlockSpec((1, D), lambda i: (0, 0)),
        ],
        out_specs=[pl.BlockSpec((BLK, D), lambda i: (i, 0)),
                   pl.BlockSpec((BLK, 1), lambda i: (i, 0))],
        out_shape=[jax.ShapeDtypeStruct((NT, D), jnp.float32),
                   jax.ShapeDtypeStruct((NT, 1), jnp.float32)],
    )(x_aug, P, P, Wlt, Wrt, b)


def _tc_layer2(h, P, invc, Wlt, Wrt, b):
    def body(h_ref, p0_ref, p1_ref, c_ref, wl_ref, wr_ref, b_ref, o_ref):
        i = pl.program_id(0)
        mean = (p0_ref[...] + p1_ref[...]) * c_ref[...]
        z = (jnp.dot(mean, wl_ref[...], preferred_element_type=jnp.float32)
             + jnp.dot(h_ref[...], wr_ref[...], preferred_element_type=jnp.float32)
             + b_ref[...])
        nrm = jnp.sqrt(jnp.sum(z * z, axis=1, keepdims=True))
        hn = jnp.maximum(z / jnp.maximum(nrm, 1e-12), 0.0)
        row = i * BLK + lax.broadcasted_iota(jnp.int32, (BLK, 1), 0)
        o_ref[...] = jnp.where(row < N, hn, 0.0)

    return pl.pallas_call(
        body,
        grid=(NT // BLK,),
        in_specs=[
            pl.BlockSpec((BLK, D), lambda i: (i, 0)),
            pl.BlockSpec((BLK, D), lambda i: (i, 0)),
            pl.BlockSpec((BLK, D), lambda i: (i + NT // BLK, 0)),
            pl.BlockSpec((BLK, 1), lambda i: (i, 0)),
            pl.BlockSpec((D, D), lambda i: (0, 0)),
            pl.BlockSpec((D, D), lambda i: (0, 0)),
            pl.BlockSpec((1, D), lambda i: (0, 0)),
        ],
        out_specs=pl.BlockSpec((BLK, D), lambda i: (i, 0)),
        out_shape=jax.ShapeDtypeStruct((NT, D), jnp.float32),
    )(h, P, P, invc, Wlt, Wrt, b)


def _tc_head(h, P, invc, W3lt, W3rt, b3, Wit, bi2, Wct, bc2, IH):
    def body(h_ref, p0_ref, p1_ref, c_ref, wl_ref, wr_ref, b3_ref, wi_ref,
             bi_ref, wc_ref, bc_ref, o_ref):
        mean = (p0_ref[...] + p1_ref[...]) * c_ref[...]
        z = (jnp.dot(mean, wl_ref[...], preferred_element_type=jnp.float32)
             + jnp.dot(h_ref[...], wr_ref[...], preferred_element_type=jnp.float32)
             + b3_ref[...])
        nrm = jnp.sqrt(jnp.sum(z * z, axis=1, keepdims=True))
        h3 = jnp.maximum(z / jnp.maximum(nrm, 1e-12), 0.0)
        h4 = jnp.maximum(
            jnp.dot(h3, wi_ref[...], preferred_element_type=jnp.float32)
            + bi_ref[...], 0.0)
        o_ref[...] = (jnp.dot(h4, wc_ref[...], preferred_element_type=jnp.float32)
                      + bc_ref[...])

    return pl.pallas_call(
        body,
        grid=(NT // BLK,),
        in_specs=[
            pl.BlockSpec((BLK, D), lambda i: (i, 0)),
            pl.BlockSpec((BLK, D), lambda i: (i, 0)),
            pl.BlockSpec((BLK, D), lambda i: (i + NT // BLK, 0)),
            pl.BlockSpec((BLK, 1), lambda i: (i, 0)),
            pl.BlockSpec((D, IH), lambda i: (0, 0)),
            pl.BlockSpec((D, IH), lambda i: (0, 0)),
            pl.BlockSpec((1, IH), lambda i: (0, 0)),
            pl.BlockSpec((IH, IH), lambda i: (0, 0)),
            pl.BlockSpec((1, IH), lambda i: (0, 0)),
            pl.BlockSpec((IH, D), lambda i: (0, 0)),
            pl.BlockSpec((1, D), lambda i: (0, 0)),
        ],
        out_specs=pl.BlockSpec((BLK, D), lambda i: (i, 0)),
        out_shape=jax.ShapeDtypeStruct((NT, D), jnp.float32),
    )(h, P, P, invc, W3lt, W3rt, b3, Wit, bi2, Wct, bc2)


def kernel(x, edge_index, batch, W1l, b1l, W1r, b1r, W2l, b2l, W2r, b2r,
           W3l, b3l, W3r, b3r, Wi, bi, Wc, bc):
    IH = Wi.shape[0]
    O = Wc.shape[0]

    x_aug = jnp.zeros((NT, DW), jnp.float32)
    x_aug = x_aug.at[:N, :D].set(x).at[:N, D].set(1.0)

    pad = N + jnp.arange(EPAD - E, dtype=jnp.int32) % (NT - N)
    src_flat = jnp.concatenate([edge_index[0], pad])
    dst_flat = jnp.concatenate([edge_index[1], pad])
    srcp_a = src_flat.reshape(NW, NCH, CH)
    dstp_a = dst_flat.reshape(NW, NCH, CH)
    srcp_b = src_flat.reshape(NW, 128, 80)
    dstp_b = dst_flat.reshape(NW, 128, 80)

    P = _sc_scatter_sum(x_aug, srcp_a, dstp_a, DW, CH, NCH)
    h1, invc = _tc_layer1(x_aug, P, W1l.T, W1r.T, (b1l + b1r)[None, :])
    P = _sc_scatter_sum(h1, srcp_b, dstp_b, D, 80, 128)
    h2 = _tc_layer2(h1, P, invc, W2l.T, W2r.T, (b2l + b2r)[None, :])
    P = _sc_scatter_sum(h2, srcp_b, dstp_b, D, 80, 128)

    Wct = jnp.zeros((D, IH), jnp.float32).at[:O].set(Wc).T
    bc2 = jnp.zeros((1, D), jnp.float32).at[0, :O].set(bc)
    out = _tc_head(h2, P, invc, W3l.T, W3r.T, (b3l + b3r)[None, :],
                   Wi.T, bi[None, :], Wct, bc2, IH)
    return out[:N, :O]


# speedup vs baseline: 1.1577x; 1.0187x over previous
import functools

import jax
import jax.numpy as jnp
from jax import lax
from jax.experimental import pallas as pl
from jax.experimental.pallas import tpu as pltpu
from jax.experimental.pallas import tpu_sc as plsc

N = 10000
NT = 10240
DW = 144
D = 128
NC = 2
NS = 16
NW = NC * NS
E = 320000
CH = 64
NB = 4
NCH = 160
EPW = NCH * CH
EPAD = NW * EPW
RPT = NT // NS
BLK = 2048


def _sc_scatter_sum(table, src_idx, dst_idx, W, CH, NCH):
    mesh = plsc.VectorSubcoreMesh(core_axis_name="c", subcore_axis_name="s",
                                  num_cores=NC, num_subcores=NS)

    @functools.partial(
        pl.kernel,
        out_type=jax.ShapeDtypeStruct((NC * NT, W), jnp.float32),
        mesh=mesh,
        scratch_types=[
            pltpu.VMEM((NB, CH), jnp.int32),
            pltpu.VMEM((NB, CH), jnp.int32),
            pltpu.VMEM((NB, CH, W), jnp.float32),
            pltpu.VMEM_SHARED((NT, W), jnp.float32),
            [pltpu.SemaphoreType.DMA] * NB,
            [pltpu.SemaphoreType.DMA] * NB,
            [pltpu.SemaphoreType.DMA] * NB,
        ],
        compiler_params=pltpu.CompilerParams(use_tc_tiling_on_sc=False),
    )
    def k(table_hbm, src_hbm, dst_hbm, out_hbm,
          sidx_r, didx_r, rows_v, acc_sh, sem_i, sem_g, sem_s):
        c = lax.axis_index("c")
        s = lax.axis_index("s")
        wid = s * NC + c

        def idx_load(j, b):
            pltpu.async_copy(src_hbm.at[wid, j], sidx_r.at[b], sem_i[b])
            pltpu.async_copy(dst_hbm.at[wid, j], didx_r.at[b], sem_i[b])

        def idx_wait(j, b):
            for _ in range(2):
                pltpu.make_async_copy(
                    src_hbm.at[wid, j], sidx_r.at[b], sem_i[b]).wait()

        def gather_start(b):
            pltpu.async_copy(table_hbm.at[sidx_r.at[b]], rows_v.at[b], sem_g[b])

        def gather_wait(b):
            pltpu.make_async_copy(
                table_hbm.at[sidx_r.at[b]], rows_v.at[b], sem_g[b]).wait()

        def scatter_start(b):
            pltpu.async_copy(rows_v.at[b], acc_sh.at[didx_r.at[b]], sem_s[b],
                             add=True)

        def scatter_wait(b):
            pltpu.make_async_copy(
                rows_v.at[b], acc_sh.at[didx_r.at[b]], sem_s[b]).wait()

        idx_load(0, 0)
        idx_load(1, 1)
        zero16 = jnp.zeros((16,), jnp.float32)
        zb = NB - 1

        def zrow(i, _):
            for j in range(W // 16):
                rows_v[zb, i, pl.ds(j * 16, 16)] = zero16
            return 0

        lax.fori_loop(0, CH, zrow, 0)
        idx_wait(0, 0)
        gather_start(0)
        zparts = [(r * CH, CH) for r in range(RPT // CH)]
        if RPT % CH:
            zparts.append((RPT - RPT % CH, RPT % CH))
        for off, ln in zparts:
            pltpu.async_copy(rows_v.at[zb, pl.ds(0, ln)],
                             acc_sh.at[pl.ds(s * RPT + off, ln)], sem_s[zb])
        for off, ln in zparts:
            pltpu.make_async_copy(
                rows_v.at[zb, pl.ds(0, ln)],
                acc_sh.at[pl.ds(s * RPT + off, ln)], sem_s[zb]).wait()
        plsc.subcore_barrier()

        def step(i, _):
            for b in range(NB):
                j = i * NB + b
                b1 = (b + 1) % NB
                b2 = (b + 2) % NB

                @pl.when(j >= 2)
                def _():
                    scatter_wait(b2)

                @pl.when(j + 2 < NCH)
                def _():
                    idx_load(j + 2, b2)

                @pl.when(j + 1 < NCH)
                def _():
                    idx_wait(j + 1, b1)
                    gather_start(b1)

                gather_wait(b)
                scatter_start(b)
            return 0

        lax.fori_loop(0, NCH // NB, step, 0)
        scatter_wait((NCH - 2) % NB)
        scatter_wait((NCH - 1) % NB)
        plsc.subcore_barrier()

        pltpu.sync_copy(acc_sh.at[pl.ds(s * RPT, RPT)],
                        out_hbm.at[pl.ds(c * NT + s * RPT, RPT)])

    return k(table, src_idx, dst_idx)


def _tc_layer1(x_aug, P, Wlt, Wrt, b):
    def body(h_ref, p0_ref, p1_ref, wl_ref, wr_ref, b_ref, o_ref, oc_ref):
        i = pl.program_id(0)
        hb = h_ref[...]
        ssum = p0_ref[...] + p1_ref[...]
        inv_c = 1.0 / jnp.maximum(ssum[:, D:D + 1], 1.0)
        mean = ssum[:, :D] * inv_c
        xr = hb[:, :D]
        z = (jnp.dot(mean, wl_ref[...], preferred_element_type=jnp.float32)
             + jnp.dot(xr, wr_ref[...], preferred_element_type=jnp.float32)
             + b_ref[...])
        nrm = jnp.sqrt(jnp.sum(z * z, axis=1, keepdims=True))
        hn = jnp.maximum(z / jnp.maximum(nrm, 1e-12), 0.0)
        row = i * BLK + lax.broadcasted_iota(jnp.int32, (BLK, 1), 0)
        valid = row < N
        o_ref[...] = jnp.where(valid, hn, 0.0)
        oc_ref[...] = jnp.where(valid, inv_c, 0.0)

    return pl.pallas_call(
        body,
        grid=(NT // BLK,),
        in_specs=[
            pl.BlockSpec((BLK, DW), lambda i: (i, 0)),
            pl.BlockSpec((BLK, DW), lambda i: (i, 0)),
            pl.BlockSpec((BLK, DW), lambda i: (i + NT // BLK, 0)),
            pl.BlockSpec((D, D), lambda i: (0, 0)),
            pl.BlockSpec((D, D), lambda i: (0, 0)),
            pl.BlockSpec((1, D), lambda i: (0, 0)),
        ],
        out_specs=[pl.BlockSpec((BLK, D), lambda i: (i, 0)),
                   pl.BlockSpec((BLK, 1), lambda i: (i, 0))],
        out_shape=[jax.ShapeDtypeStruct((NT, D), jnp.float32),
                   jax.ShapeDtypeStruct((NT, 1), jnp.float32)],
    )(x_aug, P, P, Wlt, Wrt, b)


def _tc_layer2(h, P, invc, Wlt, Wrt, b):
    def body(h_ref, p0_ref, p1_ref, c_ref, wl_ref, wr_ref, b_ref, o_ref):
        i = pl.program_id(0)
        mean = (p0_ref[...] + p1_ref[...]) * c_ref[...]
        z = (jnp.dot(mean, wl_ref[...], preferred_element_type=jnp.float32)
             + jnp.dot(h_ref[...], wr_ref[...], preferred_element_type=jnp.float32)
             + b_ref[...])
        nrm = jnp.sqrt(jnp.sum(z * z, axis=1, keepdims=True))
        hn = jnp.maximum(z / jnp.maximum(nrm, 1e-12), 0.0)
        row = i * BLK + lax.broadcasted_iota(jnp.int32, (BLK, 1), 0)
        o_ref[...] = jnp.where(row < N, hn, 0.0)

    return pl.pallas_call(
        body,
        grid=(NT // BLK,),
        in_specs=[
            pl.BlockSpec((BLK, D), lambda i: (i, 0)),
            pl.BlockSpec((BLK, D), lambda i: (i, 0)),
            pl.BlockSpec((BLK, D), lambda i: (i + NT // BLK, 0)),
            pl.BlockSpec((BLK, 1), lambda i: (i, 0)),
            pl.BlockSpec((D, D), lambda i: (0, 0)),
            pl.BlockSpec((D, D), lambda i: (0, 0)),
            pl.BlockSpec((1, D), lambda i: (0, 0)),
        ],
        out_specs=pl.BlockSpec((BLK, D), lambda i: (i, 0)),
        out_shape=jax.ShapeDtypeStruct((NT, D), jnp.float32),
    )(h, P, P, invc, Wlt, Wrt, b)


def _tc_head(h, P, invc, W3lt, W3rt, b3, Wit, bi2, Wct, bc2, IH):
    def body(h_ref, p0_ref, p1_ref, c_ref, wl_ref, wr_ref, b3_ref, wi_ref,
             bi_ref, wc_ref, bc_ref, o_ref):
        mean = (p0_ref[...] + p1_ref[...]) * c_ref[...]
        z = (jnp.dot(mean, wl_ref[...], preferred_element_type=jnp.float32)
             + jnp.dot(h_ref[...], wr_ref[...], preferred_element_type=jnp.float32)
             + b3_ref[...])
        nrm = jnp.sqrt(jnp.sum(z * z, axis=1, keepdims=True))
        h3 = jnp.maximum(z / jnp.maximum(nrm, 1e-12), 0.0)
        h4 = jnp.maximum(
            jnp.dot(h3, wi_ref[...], preferred_element_type=jnp.float32)
            + bi_ref[...], 0.0)
        o_ref[...] = (jnp.dot(h4, wc_ref[...], preferred_element_type=jnp.float32)
                      + bc_ref[...])

    return pl.pallas_call(
        body,
        grid=(NT // BLK,),
        in_specs=[
            pl.BlockSpec((BLK, D), lambda i: (i, 0)),
            pl.BlockSpec((BLK, D), lambda i: (i, 0)),
            pl.BlockSpec((BLK, D), lambda i: (i + NT // BLK, 0)),
            pl.BlockSpec((BLK, 1), lambda i: (i, 0)),
            pl.BlockSpec((D, IH), lambda i: (0, 0)),
            pl.BlockSpec((D, IH), lambda i: (0, 0)),
            pl.BlockSpec((1, IH), lambda i: (0, 0)),
            pl.BlockSpec((IH, IH), lambda i: (0, 0)),
            pl.BlockSpec((1, IH), lambda i: (0, 0)),
            pl.BlockSpec((IH, D), lambda i: (0, 0)),
            pl.BlockSpec((1, D), lambda i: (0, 0)),
        ],
        out_specs=pl.BlockSpec((BLK, D), lambda i: (i, 0)),
        out_shape=jax.ShapeDtypeStruct((NT, D), jnp.float32),
    )(h, P, P, invc, W3lt, W3rt, b3, Wit, bi2, Wct, bc2)


def kernel(x, edge_index, batch, W1l, b1l, W1r, b1r, W2l, b2l, W2r, b2r,
           W3l, b3l, W3r, b3r, Wi, bi, Wc, bc):
    IH = Wi.shape[0]
    O = Wc.shape[0]

    x_aug = jnp.zeros((NT, DW), jnp.float32)
    x_aug = x_aug.at[:N, :D].set(x).at[:N, D].set(1.0)

    pad = N + jnp.arange(EPAD - E, dtype=jnp.int32) % (NT - N)
    src_flat = jnp.concatenate([edge_index[0], pad])
    dst_flat = jnp.concatenate([edge_index[1], pad])
    srcp_a = src_flat.reshape(NW, NCH, CH)
    dstp_a = dst_flat.reshape(NW, NCH, CH)
    srcp_b = src_flat[:NW * 116 * 88].reshape(NW, 116, 88)
    dstp_b = dst_flat[:NW * 116 * 88].reshape(NW, 116, 88)

    P = _sc_scatter_sum(x_aug, srcp_a, dstp_a, DW, CH, NCH)
    h1, invc = _tc_layer1(x_aug, P, W1l.T, W1r.T, (b1l + b1r)[None, :])
    P = _sc_scatter_sum(h1, srcp_b, dstp_b, D, 88, 116)
    h2 = _tc_layer2(h1, P, invc, W2l.T, W2r.T, (b2l + b2r)[None, :])
    P = _sc_scatter_sum(h2, srcp_b, dstp_b, D, 88, 116)

    Wct = jnp.zeros((D, IH), jnp.float32).at[:O].set(Wc).T
    bc2 = jnp.zeros((1, D), jnp.float32).at[0, :O].set(bc)
    out = _tc_head(h2, P, invc, W3l.T, W3r.T, (b3l + b3r)[None, :],
                   Wi.T, bi[None, :], Wct, bc2, IH)
    return out[:N, :O]


# speedup vs baseline: 1.3165x; 1.1371x over previous
import functools

import jax
import jax.numpy as jnp
from jax import lax
from jax.experimental import pallas as pl
from jax.experimental.pallas import tpu as pltpu
from jax.experimental.pallas import tpu_sc as plsc

N = 10000
NT = 10240
D = 128
NC = 2
NS = 16
NW = NC * NS
E = 320000
NB = 4
EPAD = NW * 10368
RPT = NT // NS
BLK = 2048


def _sc_scatter_sum(table, src_idx, dst_idx, W, CH, NCH, with_counts=False):
    mesh = plsc.VectorSubcoreMesh(core_axis_name="c", subcore_axis_name="s",
                                  num_cores=NC, num_subcores=NS)
    CW = 16
    out_type = [jax.ShapeDtypeStruct((NC * NT, W), jnp.float32)]
    cnt_scratch = []
    if with_counts:
        out_type.append(jax.ShapeDtypeStruct((NC * NT, CW), jnp.float32))
        cnt_scratch = [pltpu.VMEM((CH, CW), jnp.float32),
                       pltpu.VMEM_SHARED((NT, CW), jnp.float32)]

    @functools.partial(
        pl.kernel,
        out_type=out_type,
        mesh=mesh,
        scratch_types=[
            pltpu.VMEM((NB, CH), jnp.int32),
            pltpu.VMEM((NB, CH), jnp.int32),
            pltpu.VMEM((NB, CH, W), jnp.float32),
            pltpu.VMEM_SHARED((NT, W), jnp.float32),
            *cnt_scratch,
            [pltpu.SemaphoreType.DMA] * NB,
            [pltpu.SemaphoreType.DMA] * NB,
            [pltpu.SemaphoreType.DMA] * NB,
        ],
        compiler_params=pltpu.CompilerParams(use_tc_tiling_on_sc=False),
    )
    def k(table_hbm, src_hbm, dst_hbm, out_hbm, *rest):
        if with_counts:
            (cnt_hbm, sidx_r, didx_r, rows_v, acc_sh, ones_v, cnt_sh,
             sem_i, sem_g, sem_s) = rest
        else:
            sidx_r, didx_r, rows_v, acc_sh, sem_i, sem_g, sem_s = rest
        c = lax.axis_index("c")
        s = lax.axis_index("s")
        wid = s * NC + c

        def idx_load(j, b):
            pltpu.async_copy(src_hbm.at[wid, j], sidx_r.at[b], sem_i[b])
            pltpu.async_copy(dst_hbm.at[wid, j], didx_r.at[b], sem_i[b])

        def idx_wait(j, b):
            for _ in range(2):
                pltpu.make_async_copy(
                    src_hbm.at[wid, j], sidx_r.at[b], sem_i[b]).wait()

        def gather_start(b):
            pltpu.async_copy(table_hbm.at[sidx_r.at[b]], rows_v.at[b], sem_g[b])

        def gather_wait(b):
            pltpu.make_async_copy(
                table_hbm.at[sidx_r.at[b]], rows_v.at[b], sem_g[b]).wait()

        def scatter_start(b):
            pltpu.async_copy(rows_v.at[b], acc_sh.at[didx_r.at[b]], sem_s[b],
                             add=True)
            if with_counts:
                pltpu.async_copy(ones_v, cnt_sh.at[didx_r.at[b]], sem_s[b],
                                 add=True)

        def scatter_wait(b):
            pltpu.make_async_copy(
                rows_v.at[b], acc_sh.at[didx_r.at[b]], sem_s[b]).wait()
            if with_counts:
                pltpu.make_async_copy(
                    ones_v, cnt_sh.at[didx_r.at[b]], sem_s[b]).wait()

        idx_load(0, 0)
        idx_load(1, 1)
        zero16 = jnp.zeros((16,), jnp.float32)
        zb = NB - 1

        def zrow(i, _):
            for j in range(W // 16):
                rows_v[zb, i, pl.ds(j * 16, 16)] = zero16
            return 0

        if with_counts:
            def czrow(i, _):
                ones_v[i, pl.ds(0, 16)] = zero16
                return 0

            lax.fori_loop(0, CH, czrow, 0)
        lax.fori_loop(0, CH, zrow, 0)
        idx_wait(0, 0)
        gather_start(0)
        zparts = [(r * CH, CH) for r in range(RPT // CH)]
        if RPT % CH:
            zparts.append((RPT - RPT % CH, RPT % CH))
        for off, ln in zparts:
            pltpu.async_copy(rows_v.at[zb, pl.ds(0, ln)],
                             acc_sh.at[pl.ds(s * RPT + off, ln)], sem_s[zb])
            if with_counts:
                pltpu.async_copy(ones_v.at[pl.ds(0, ln)],
                                 cnt_sh.at[pl.ds(s * RPT + off, ln)], sem_s[zb])
        for off, ln in zparts:
            pltpu.make_async_copy(
                rows_v.at[zb, pl.ds(0, ln)],
                acc_sh.at[pl.ds(s * RPT + off, ln)], sem_s[zb]).wait()
            if with_counts:
                pltpu.make_async_copy(
                    ones_v.at[pl.ds(0, ln)],
                    cnt_sh.at[pl.ds(s * RPT + off, ln)], sem_s[zb]).wait()
        if with_counts:
            one16 = jnp.ones((16,), jnp.float32)

            def orow(i, _):
                ones_v[i, pl.ds(0, 16)] = one16
                return 0

            lax.fori_loop(0, CH, orow, 0)
        plsc.subcore_barrier()

        def step(i, _):
            for b in range(NB):
                j = i * NB + b
                b1 = (b + 1) % NB
                b2 = (b + 2) % NB

                @pl.when(j >= 2)
                def _():
                    scatter_wait(b2)

                @pl.when(j + 2 < NCH)
                def _():
                    idx_load(j + 2, b2)

                @pl.when(j + 1 < NCH)
                def _():
                    idx_wait(j + 1, b1)
                    gather_start(b1)

                gather_wait(b)
                scatter_start(b)
            return 0

        lax.fori_loop(0, NCH // NB, step, 0)
        scatter_wait((NCH - 2) % NB)
        scatter_wait((NCH - 1) % NB)
        plsc.subcore_barrier()

        pltpu.sync_copy(acc_sh.at[pl.ds(s * RPT, RPT)],
                        out_hbm.at[pl.ds(c * NT + s * RPT, RPT)])
        if with_counts:
            pltpu.sync_copy(cnt_sh.at[pl.ds(s * RPT, RPT)],
                            cnt_hbm.at[pl.ds(c * NT + s * RPT, RPT)])

    return k(table, src_idx, dst_idx)


def _tc_layer1(x_pad, P, CNT, Wlt, Wrt, b):
    def body(h_ref, p0_ref, p1_ref, c0_ref, c1_ref, wl_ref, wr_ref, b_ref,
             o_ref, oc_ref):
        i = pl.program_id(0)
        counts = (c0_ref[...] + c1_ref[...])[:, :1]
        inv_c = 1.0 / jnp.maximum(counts, 1.0)
        mean = (p0_ref[...] + p1_ref[...]) * inv_c
        z = (jnp.dot(mean, wl_ref[...], preferred_element_type=jnp.float32)
             + jnp.dot(h_ref[...], wr_ref[...], preferred_element_type=jnp.float32)
             + b_ref[...])
        nrm = jnp.sqrt(jnp.sum(z * z, axis=1, keepdims=True))
        hn = jnp.maximum(z / jnp.maximum(nrm, 1e-12), 0.0)
        row = i * BLK + lax.broadcasted_iota(jnp.int32, (BLK, 1), 0)
        valid = row < N
        o_ref[...] = jnp.where(valid, hn, 0.0)
        oc_ref[...] = jnp.where(valid, inv_c, 0.0)

    return pl.pallas_call(
        body,
        grid=(NT // BLK,),
        in_specs=[
            pl.BlockSpec((BLK, D), lambda i: (i, 0)),
            pl.BlockSpec((BLK, D), lambda i: (i, 0)),
            pl.BlockSpec((BLK, D), lambda i: (i + NT // BLK, 0)),
            pl.BlockSpec((BLK, 16), lambda i: (i, 0)),
            pl.BlockSpec((BLK, 16), lambda i: (i + NT // BLK, 0)),
            pl.BlockSpec((D, D), lambda i: (0, 0)),
            pl.BlockSpec((D, D), lambda i: (0, 0)),
            pl.BlockSpec((1, D), lambda i: (0, 0)),
        ],
        out_specs=[pl.BlockSpec((BLK, D), lambda i: (i, 0)),
                   pl.BlockSpec((BLK, 1), lambda i: (i, 0))],
        out_shape=[jax.ShapeDtypeStruct((NT, D), jnp.float32),
                   jax.ShapeDtypeStruct((NT, 1), jnp.float32)],
    )(x_pad, P, P, CNT, CNT, Wlt, Wrt, b)


def _tc_layer2(h, P, invc, Wlt, Wrt, b):
    def body(h_ref, p0_ref, p1_ref, c_ref, wl_ref, wr_ref, b_ref, o_ref):
        i = pl.program_id(0)
        mean = (p0_ref[...] + p1_ref[...]) * c_ref[...]
        z = (jnp.dot(mean, wl_ref[...], preferred_element_type=jnp.float32)
             + jnp.dot(h_ref[...], wr_ref[...], preferred_element_type=jnp.float32)
             + b_ref[...])
        nrm = jnp.sqrt(jnp.sum(z * z, axis=1, keepdims=True))
        hn = jnp.maximum(z / jnp.maximum(nrm, 1e-12), 0.0)
        row = i * BLK + lax.broadcasted_iota(jnp.int32, (BLK, 1), 0)
        o_ref[...] = jnp.where(row < N, hn, 0.0)

    return pl.pallas_call(
        body,
        grid=(NT // BLK,),
        in_specs=[
            pl.BlockSpec((BLK, D), lambda i: (i, 0)),
            pl.BlockSpec((BLK, D), lambda i: (i, 0)),
            pl.BlockSpec((BLK, D), lambda i: (i + NT // BLK, 0)),
            pl.BlockSpec((BLK, 1), lambda i: (i, 0)),
            pl.BlockSpec((D, D), lambda i: (0, 0)),
            pl.BlockSpec((D, D), lambda i: (0, 0)),
            pl.BlockSpec((1, D), lambda i: (0, 0)),
        ],
        out_specs=pl.BlockSpec((BLK, D), lambda i: (i, 0)),
        out_shape=jax.ShapeDtypeStruct((NT, D), jnp.float32),
    )(h, P, P, invc, Wlt, Wrt, b)


def _tc_head(h, P, invc, W3lt, W3rt, b3, Wit, bi2, Wct, bc2, IH):
    def body(h_ref, p0_ref, p1_ref, c_ref, wl_ref, wr_ref, b3_ref, wi_ref,
             bi_ref, wc_ref, bc_ref, o_ref):
        mean = (p0_ref[...] + p1_ref[...]) * c_ref[...]
        z = (jnp.dot(mean, wl_ref[...], preferred_element_type=jnp.float32)
             + jnp.dot(h_ref[...], wr_ref[...], preferred_element_type=jnp.float32)
             + b3_ref[...])
        nrm = jnp.sqrt(jnp.sum(z * z, axis=1, keepdims=True))
        h3 = jnp.maximum(z / jnp.maximum(nrm, 1e-12), 0.0)
        h4 = jnp.maximum(
            jnp.dot(h3, wi_ref[...], preferred_element_type=jnp.float32)
            + bi_ref[...], 0.0)
        o_ref[...] = (jnp.dot(h4, wc_ref[...], preferred_element_type=jnp.float32)
                      + bc_ref[...])

    return pl.pallas_call(
        body,
        grid=(NT // BLK,),
        in_specs=[
            pl.BlockSpec((BLK, D), lambda i: (i, 0)),
            pl.BlockSpec((BLK, D), lambda i: (i, 0)),
            pl.BlockSpec((BLK, D), lambda i: (i + NT // BLK, 0)),
            pl.BlockSpec((BLK, 1), lambda i: (i, 0)),
            pl.BlockSpec((D, IH), lambda i: (0, 0)),
            pl.BlockSpec((D, IH), lambda i: (0, 0)),
            pl.BlockSpec((1, IH), lambda i: (0, 0)),
            pl.BlockSpec((IH, IH), lambda i: (0, 0)),
            pl.BlockSpec((1, IH), lambda i: (0, 0)),
            pl.BlockSpec((IH, D), lambda i: (0, 0)),
            pl.BlockSpec((1, D), lambda i: (0, 0)),
        ],
        out_specs=pl.BlockSpec((BLK, D), lambda i: (i, 0)),
        out_shape=jax.ShapeDtypeStruct((NT, D), jnp.float32),
    )(h, P, P, invc, W3lt, W3rt, b3, Wit, bi2, Wct, bc2)


def kernel(x, edge_index, batch, W1l, b1l, W1r, b1r, W2l, b2l, W2r, b2r,
           W3l, b3l, W3r, b3r, Wi, bi, Wc, bc):
    IH = Wi.shape[0]
    O = Wc.shape[0]

    x_pad = jnp.pad(x, ((0, NT - N), (0, 0)))

    pad = N + jnp.arange(EPAD - E, dtype=jnp.int32) % (NT - N)
    src_flat = jnp.concatenate([edge_index[0], pad])
    dst_flat = jnp.concatenate([edge_index[1], pad])
    srcp_a = src_flat.reshape(NW, 144, 72)
    dstp_a = dst_flat.reshape(NW, 144, 72)
    srcp_b = src_flat[:NW * 116 * 88].reshape(NW, 116, 88)
    dstp_b = dst_flat[:NW * 116 * 88].reshape(NW, 116, 88)

    P, CNT = _sc_scatter_sum(x_pad, srcp_a, dstp_a, D, 72, 144,
                             with_counts=True)
    h1, invc = _tc_layer1(x_pad, P, CNT, W1l.T, W1r.T, (b1l + b1r)[None, :])
    P, = _sc_scatter_sum(h1, srcp_b, dstp_b, D, 88, 116)
    h2 = _tc_layer2(h1, P, invc, W2l.T, W2r.T, (b2l + b2r)[None, :])
    P, = _sc_scatter_sum(h2, srcp_b, dstp_b, D, 88, 116)

    Wct = jnp.zeros((D, IH), jnp.float32).at[:O].set(Wc).T
    bc2 = jnp.zeros((1, D), jnp.float32).at[0, :O].set(bc)
    out = _tc_head(h2, P, invc, W3l.T, W3r.T, (b3l + b3r)[None, :],
                   Wi.T, bi[None, :], Wct, bc2, IH)
    return out[:N, :O]
